# baseline (speedup 1.0000x reference)
import functools

import jax
import jax.numpy as jnp
from jax import lax
from jax.experimental import pallas as pl
from jax.experimental.pallas import tpu as pltpu
from jax.experimental.pallas import tpu_sc as plsc

_NC = 2
_NS = 16
_NW = _NC * _NS

_GRP = 128


def kernel(idxes, table):
    B, H = idxes.shape
    V, D = table.shape

    idx_t = idxes.T

    BW = B // _NW
    G = BW // _GRP
    LANES = 16
    assert BW * _NW == B and G * _GRP == BW and H % 2 == 0 and H >= 6

    mesh = plsc.VectorSubcoreMesh(core_axis_name="c", subcore_axis_name="s")

    @functools.partial(
        pl.kernel,
        mesh=mesh,
        out_type=jax.ShapeDtypeStruct((H * D, B), jnp.float32),
        compiler_params=pltpu.CompilerParams(
            use_tc_tiling_on_sc=False, needs_layout_passes=False),
        scratch_types=[
            pltpu.VMEM((G, _GRP), jnp.int32),
            pltpu.VMEM((G, _GRP), jnp.int32),
            pltpu.VMEM((BW, D), jnp.float32),
            pltpu.VMEM((BW, D), jnp.float32),
            pltpu.VMEM((D, BW), jnp.float32),
            pltpu.VMEM((D, BW), jnp.float32),
            pltpu.SemaphoreType.DMA,
            pltpu.SemaphoreType.DMA,
            pltpu.SemaphoreType.DMA,
            pltpu.SemaphoreType.DMA,
            pltpu.SemaphoreType.DMA,
            pltpu.SemaphoreType.DMA,
        ],
    )
    def run(idx_hbm, table_hbm, out_hbm,
            idx0, idx1, rows0, rows1, rt0, rt1,
            si0, si1, sg0, sg1, so0, so1):
        idx_v = (idx0, idx1)
        rows_v = (rows0, rows1)
        rt_v = (rt0, rt1)
        semi = (si0, si1)
        semg = (sg0, sg1)
        semo = (so0, so1)

        wid = lax.axis_index("s") * _NC + lax.axis_index("c")
        b0 = wid * BW

        def fire_idx(h, nb):
            for g in range(G):
                pltpu.async_copy(
                    idx_hbm.at[pl.ds(h, 1), pl.ds(b0 + g * _GRP, _GRP)],
                    idx_v[nb].at[pl.ds(g, 1)],
                    semi[nb])

        def wait_idx(nb):
            pltpu.make_async_copy(
                idx_hbm.at[pl.ds(0, G), pl.ds(0, _GRP)], idx_v[nb],
                semi[nb]).wait()

        def fire_gathers(nb):
            for g in range(G):
                pltpu.async_copy(
                    table_hbm.at[idx_v[nb].at[g]],
                    rows_v[nb].at[pl.ds(g * _GRP, _GRP)],
                    semg[nb])

        def wait_gathers(nb):
            pltpu.make_async_copy(
                table_hbm.at[pl.ds(0, BW)], rows_v[nb], semg[nb]).wait()

        lane_iota = jnp.arange(LANES, dtype=jnp.int32)
        col_ids = [jnp.full((LANES,), c, dtype=jnp.int32) for c in range(D)]

        def transpose(nb):
            rows = rows_v[nb]
            rt = rt_v[nb]

            @plsc.parallel_loop(0, BW // LANES, unroll=4)
            def tbody(lb):
                row_ids = lb * LANES + lane_iota
                for c in range(D):
                    v = plsc.load_gather(rows, [row_ids, col_ids[c]])
                    rt[c, pl.ds(lb * LANES, LANES)] = v

        def fire_out(h, nb):
            pltpu.async_copy(
                rt_v[nb], out_hbm.at[pl.ds(h * D, D), pl.ds(b0, BW)],
                semo[nb])

        def wait_out(nb):
            pltpu.make_async_copy(
                rt_v[nb], out_hbm.at[pl.ds(0, D), pl.ds(0, BW)],
                semo[nb]).wait()

        def step(h, b, pb, first_use):
            wait_idx(b)
            fire_gathers(b)
            wait_gathers(pb)
            if not first_use:
                wait_out(pb)
            transpose(pb)
            fire_out(h - 1, pb)
            fire_idx(h + 1, pb)

        fire_idx(0, 0)
        fire_idx(1, 1)
        wait_idx(0)
        fire_gathers(0)
        step(1, 1, 0, first_use=True)
        step(2, 0, 1, first_use=True)
        step(3, 1, 0, first_use=False)

        def body(g, carry):
            h = 2 * g
            step(h, 0, 1, first_use=False)
            step(h + 1, 1, 0, first_use=False)
            return carry

        lax.fori_loop(2, H // 2 - 1, body, 0)

        step(H - 2, 0, 1, first_use=False)
        wait_idx(1)
        fire_gathers(1)
        wait_gathers(0)
        wait_out(0)
        transpose(0)
        fire_out(H - 2, 0)
        wait_gathers(1)
        wait_out(1)
        transpose(1)
        fire_out(H - 1, 1)
        wait_out(0)
        wait_out(1)

    out = run(idx_t, table)
    return jnp.transpose(out.reshape(H, D, B), (2, 0, 1))


# speedup vs baseline: 1.0822x; 1.0822x over previous
import functools

import jax
import jax.numpy as jnp
from jax import lax
from jax.experimental import pallas as pl
from jax.experimental.pallas import tpu as pltpu
from jax.experimental.pallas import tpu_sc as plsc

_NC = 2
_NS = 16
_NW = _NC * _NS

_GRP = 128


def kernel(idxes, table):
    B, H = idxes.shape
    V, D = table.shape

    idx_t = idxes.T

    BW = B // _NW
    G = BW // _GRP
    LANES = 16
    assert BW * _NW == B and G * _GRP == BW and H % 2 == 0 and H >= 6

    mesh = plsc.VectorSubcoreMesh(core_axis_name="c", subcore_axis_name="s")

    @functools.partial(
        pl.kernel,
        mesh=mesh,
        out_type=jax.ShapeDtypeStruct((H * D, B), jnp.float32),
        compiler_params=pltpu.CompilerParams(
            use_tc_tiling_on_sc=False, needs_layout_passes=False),
        scratch_types=[
            pltpu.VMEM((G, _GRP), jnp.int32),
            pltpu.VMEM((G, _GRP), jnp.int32),
            pltpu.VMEM((BW, D), jnp.float32),
            pltpu.VMEM((BW, D), jnp.float32),
            pltpu.VMEM((D, BW), jnp.float32),
            pltpu.VMEM((D, BW), jnp.float32),
            pltpu.SemaphoreType.DMA,
            pltpu.SemaphoreType.DMA,
            pltpu.SemaphoreType.DMA,
            pltpu.SemaphoreType.DMA,
            pltpu.SemaphoreType.DMA,
            pltpu.SemaphoreType.DMA,
        ],
    )
    def run(idx_hbm, table_hbm, out_hbm,
            idx0, idx1, rows0, rows1, rt0, rt1,
            si0, si1, sg0, sg1, so0, so1):
        idx_v = (idx0, idx1)
        rows_v = (rows0, rows1)
        rt_v = (rt0, rt1)
        semi = (si0, si1)
        semg = (sg0, sg1)
        semo = (so0, so1)

        wid = lax.axis_index("s") * _NC + lax.axis_index("c")
        b0 = wid * BW

        def fire_idx(h, nb):
            for g in range(G):
                pltpu.async_copy(
                    idx_hbm.at[pl.ds(h, 1), pl.ds(b0 + g * _GRP, _GRP)],
                    idx_v[nb].at[pl.ds(g, 1)],
                    semi[nb])

        def wait_idx(nb):
            pltpu.make_async_copy(
                idx_hbm.at[pl.ds(0, G), pl.ds(0, _GRP)], idx_v[nb],
                semi[nb]).wait()

        def fire_gathers(nb):
            for g in range(G):
                pltpu.async_copy(
                    table_hbm.at[idx_v[nb].at[g]],
                    rows_v[nb].at[pl.ds(g * _GRP, _GRP)],
                    semg[nb])

        def wait_gathers(nb):
            pltpu.make_async_copy(
                table_hbm.at[pl.ds(0, BW)], rows_v[nb], semg[nb]).wait()

        lane_iota = jnp.arange(LANES, dtype=jnp.int32)
        col_ids = [jnp.full((LANES,), c, dtype=jnp.int32) for c in range(D)]

        def transpose(nb):
            rows = rows_v[nb]
            rt = rt_v[nb]

            @plsc.parallel_loop(0, BW // LANES)
            def tbody(lb):
                row_ids = lb * LANES + lane_iota
                for c in range(D):
                    v = plsc.load_gather(rows, [row_ids, col_ids[c]])
                    rt[c, pl.ds(lb * LANES, LANES)] = v

        def fire_out(h, nb):
            pltpu.async_copy(
                rt_v[nb], out_hbm.at[pl.ds(h * D, D), pl.ds(b0, BW)],
                semo[nb])

        def wait_out(nb):
            pltpu.make_async_copy(
                rt_v[nb], out_hbm.at[pl.ds(0, D), pl.ds(0, BW)],
                semo[nb]).wait()

        def step(h, b, pb, first_use):
            wait_idx(b)
            fire_gathers(b)
            wait_gathers(pb)
            if not first_use:
                wait_out(pb)
            transpose(pb)
            fire_out(h - 1, pb)
            fire_idx(h + 1, pb)

        fire_idx(0, 0)
        fire_idx(1, 1)
        wait_idx(0)
        fire_gathers(0)
        step(1, 1, 0, first_use=True)
        step(2, 0, 1, first_use=True)
        step(3, 1, 0, first_use=False)

        def body(g, carry):
            h = 2 * g
            step(h, 0, 1, first_use=False)
            step(h + 1, 1, 0, first_use=False)
            return carry

        lax.fori_loop(2, H // 2 - 1, body, 0)

        step(H - 2, 0, 1, first_use=False)
        wait_idx(1)
        fire_gathers(1)
        wait_gathers(0)
        wait_out(0)
        transpose(0)
        fire_out(H - 2, 0)
        wait_gathers(1)
        wait_out(1)
        transpose(1)
        fire_out(H - 1, 1)
        wait_out(0)
        wait_out(1)

    out = run(idx_t, table)
    return jnp.transpose(out.reshape(H, D, B), (2, 0, 1))
